# baseline (speedup 1.0000x reference)
import functools

import jax
import jax.numpy as jnp
from jax import lax
from jax.experimental import pallas as pl
from jax.experimental.pallas import tpu as pltpu
from jax.experimental.pallas import tpu_sc as plsc

N = 10000
D = 128
G = 64
NC = 2
NS = 16
NW = NC * NS
C = 64
BLK = 8
NPAD = 10240
RPT = NPAD // NS
LANES = 16
DX = D + LANES


def _dense_head(h, a_src, a_dst, hx_ref, as_ref, ad_ref, bbound_ref):
    hx_ref[...] = jnp.concatenate([h, jnp.ones((N, LANES), jnp.float32)], 1)
    as_ = jnp.dot(h, a_src, preferred_element_type=jnp.float32)
    ad_ = jnp.dot(h, a_dst, preferred_element_type=jnp.float32)
    pad = jnp.zeros((NPAD - N,), jnp.float32)
    as_ref[...] = jnp.broadcast_to(
        jnp.concatenate([as_, pad])[:, None], (NPAD, LANES))
    ad_ref[...] = jnp.broadcast_to(
        jnp.concatenate([ad_, pad])[:, None], (NPAD, LANES))
    m = jnp.max(as_) + jnp.max(ad_)
    b = jnp.where(m >= 0.0, m, 0.2 * m)
    bbound_ref[...] = jnp.full((LANES,), b, jnp.float32)


def _tc_pre_body(x_ref, w_ref, asrc_ref, adst_ref, hx_ref, as_ref, ad_ref,
                 bb_ref):
    h = jnp.dot(x_ref[...], w_ref[...], preferred_element_type=jnp.float32)
    _dense_head(h, asrc_ref[...], adst_ref[...], hx_ref, as_ref, ad_ref,
                bb_ref)


def _combine_bn_relu(acc_ref, b, gamma, beta):
    nm = acc_ref[0, pl.ds(0, N), pl.ds(0, D)] + acc_ref[1, pl.ds(0, N), pl.ds(0, D)]
    dn = acc_ref[0, pl.ds(0, N), D] + acc_ref[1, pl.ds(0, N), D]
    o = nm / (dn + 1e-16)[:, None] + b[None, :]
    mu = jnp.mean(o, axis=0)
    var = jnp.mean((o - mu[None, :]) ** 2, axis=0)
    xn = (o - mu[None, :]) / jnp.sqrt(var + 1e-5) * gamma[None, :] + beta[None, :]
    return jnp.maximum(xn, 0.0)


def _tc_mid_body(acc_ref, b_ref, g_ref, be_ref, w_ref, asrc_ref,
                 adst_ref, hx_ref, as_ref, ad_ref, bb_ref):
    x2 = _combine_bn_relu(acc_ref, b_ref[...], g_ref[...], be_ref[...])
    h = jnp.dot(x2, w_ref[...], preferred_element_type=jnp.float32)
    _dense_head(h, asrc_ref[...], adst_ref[...], hx_ref, as_ref, ad_ref,
                bb_ref)


def _tc_final_body(acc_ref, b_ref, g_ref, be_ref, batch_ref, out_ref):
    hf = _combine_bn_relu(acc_ref, b_ref[...], g_ref[...], be_ref[...])
    bi = batch_ref[...]
    gids = lax.broadcasted_iota(jnp.int32, (N, G), 1)
    oh = (bi == gids).astype(jnp.float32)
    pooled = lax.dot_general(oh, hf, (((0,), (0,)), ((), ())),
                             preferred_element_type=jnp.float32)
    counts = jnp.sum(oh, axis=0)
    out_ref[...] = pooled / jnp.maximum(counts, 1.0)[:, None]


def _sc_edge_body(nb, hx_hbm, as_hbm, ad_hbm, bb_hbm, src_hbm, dst_hbm,
                  acc_out,
                  acc_sh, srcb_v, dstb_v, rows0_v, rows1_v, asr0_v, asr1_v,
                  adr0_v, adr1_v, b_v, sem0, sem1):
    cid = lax.axis_index("c")
    sid = lax.axis_index("s")
    wid = cid * NS + sid

    rows = (rows0_v, rows1_v)
    asr = (asr0_v, asr1_v)
    adr = (adr0_v, adr1_v)
    sems = (sem0, sem1)

    pltpu.sync_copy(bb_hbm, b_v)

    zero16 = jnp.zeros((LANES,), jnp.float32)

    def zero_body(i, _):
        for k in range(DX // LANES):
            rows0_v[i, pl.ds(k * LANES, LANES)] = zero16
        return 0

    lax.fori_loop(0, C, zero_body, 0)
    for r in range(RPT // C):
        pltpu.sync_copy(rows0_v, acc_sh.at[pl.ds(sid * RPT + r * C, C)])
    plsc.subcore_barrier()

    bshift = b_v[...]

    descs = [None, None]

    def issue(jj, buf):
        descs[buf] = (
            pltpu.async_copy(as_hbm.at[srcb_v.at[jj]], asr[buf], sems[buf]),
            pltpu.async_copy(ad_hbm.at[dstb_v.at[jj]], adr[buf], sems[buf]),
            pltpu.async_copy(hx_hbm.at[srcb_v.at[jj]], rows[buf], sems[buf]),
        )

    def drain(buf):
        for d in descs[buf]:
            d.wait()

    def scale_and_scatter(jj, buf):
        def scale_body(i, _):
            a = asr[buf][i, :] + adr[buf][i, :]
            a = jnp.where(a >= 0.0, a, 0.2 * a)
            ev = jnp.exp(a - bshift)
            for k in range(DX // LANES):
                rows[buf][i, pl.ds(k * LANES, LANES)] = (
                    rows[buf][i, pl.ds(k * LANES, LANES)] * ev)
            return 0

        lax.fori_loop(0, C, scale_body, 0)
        pltpu.sync_copy(rows[buf], acc_sh.at[dstb_v.at[jj]], add=True)

    def block_body(b, _):
        pltpu.sync_copy(src_hbm.at[wid, b], srcb_v)
        pltpu.sync_copy(dst_hbm.at[wid, b], dstb_v)
        issue(0, 0)
        for jj in range(BLK):
            cur = jj % 2
            drain(cur)
            if jj + 1 < BLK:
                issue(jj + 1, 1 - cur)
            scale_and_scatter(jj, cur)
        return 0

    lax.fori_loop(0, nb, block_body, 0)
    plsc.subcore_barrier()

    for r in range(RPT // C):
        buf = rows[r % 2]
        pltpu.sync_copy(acc_sh.at[pl.ds(sid * RPT + r * C, C)], buf)
        pltpu.sync_copy(buf, acc_out.at[cid, pl.ds(sid * RPT + r * C, C)])


def _sc_edge_pass(hx, as_, ad_, bb, src2d, dst2d, nb):
    mesh = plsc.VectorSubcoreMesh(core_axis_name="c", subcore_axis_name="s",
                                  num_cores=NC, num_subcores=NS)
    kern = pl.kernel(
        functools.partial(_sc_edge_body, nb),
        out_type=jax.ShapeDtypeStruct((NC, NPAD, DX), jnp.float32),
        mesh=mesh,
        compiler_params=pltpu.CompilerParams(needs_layout_passes=False,
                                             use_tc_tiling_on_sc=False),
        scratch_types=[
            pltpu.VMEM_SHARED((NPAD, DX), jnp.float32),
            pltpu.VMEM((BLK, C), jnp.int32),
            pltpu.VMEM((BLK, C), jnp.int32),
            pltpu.VMEM((C, DX), jnp.float32),
            pltpu.VMEM((C, DX), jnp.float32),
            pltpu.VMEM((C, LANES), jnp.float32),
            pltpu.VMEM((C, LANES), jnp.float32),
            pltpu.VMEM((C, LANES), jnp.float32),
            pltpu.VMEM((C, LANES), jnp.float32),
            pltpu.VMEM((LANES,), jnp.float32),
            pltpu.SemaphoreType.DMA,
            pltpu.SemaphoreType.DMA,
        ],
    )
    return kern(hx, as_, ad_, bb, src2d, dst2d)


def kernel(x, params, edge_index, batch):
    e = edge_index.shape[1]
    etot = e + N
    nb = -(-etot // (NW * C * BLK))
    epad = NW * nb * BLK * C - etot

    loop = jnp.arange(N, dtype=jnp.int32)
    src2d = jnp.concatenate(
        [edge_index[0], loop, jnp.zeros((epad,), jnp.int32)]
    ).reshape(NW, nb, BLK, C)
    dst2d = jnp.concatenate(
        [edge_index[1], loop, jnp.full((epad,), N, jnp.int32)]
    ).reshape(NW, nb, BLK, C)
    batch2 = batch.reshape(N, 1)

    tc_pre = pl.pallas_call(
        _tc_pre_body,
        out_shape=[
            jax.ShapeDtypeStruct((N, DX), jnp.float32),
            jax.ShapeDtypeStruct((NPAD, LANES), jnp.float32),
            jax.ShapeDtypeStruct((NPAD, LANES), jnp.float32),
            jax.ShapeDtypeStruct((LANES,), jnp.float32),
        ],
    )
    tc_mid = pl.pallas_call(
        _tc_mid_body,
        out_shape=[
            jax.ShapeDtypeStruct((N, DX), jnp.float32),
            jax.ShapeDtypeStruct((NPAD, LANES), jnp.float32),
            jax.ShapeDtypeStruct((NPAD, LANES), jnp.float32),
            jax.ShapeDtypeStruct((LANES,), jnp.float32),
        ],
    )
    tc_final = pl.pallas_call(
        _tc_final_body,
        out_shape=jax.ShapeDtypeStruct((G, D), jnp.float32),
    )

    p = params
    hx1, as1, ad1, bb1 = tc_pre(x, p['W1'], p['a_src1'], p['a_dst1'])
    acc1 = _sc_edge_pass(hx1, as1, ad1, bb1, src2d, dst2d, nb)
    hx2, as2, ad2, bb2 = tc_mid(acc1, p['b1'], p['gamma1'], p['beta1'],
                                p['W2'], p['a_src2'], p['a_dst2'])
    acc2 = _sc_edge_pass(hx2, as2, ad2, bb2, src2d, dst2d, nb)
    return tc_final(acc2, p['b2'], p['gamma2'], p['beta2'], batch2)


# speedup vs baseline: 1.3742x; 1.3742x over previous
import functools

import jax
import jax.numpy as jnp
from jax import lax
from jax.experimental import pallas as pl
from jax.experimental.pallas import tpu as pltpu
from jax.experimental.pallas import tpu_sc as plsc

N = 10000
D = 128
G = 64
NC = 2
NS = 16
NW = NC * NS
C = 128
BLK = 2
NPAD = 10240
RPT = NPAD // NS
LANES = 16
DX = D + LANES


def _dense_head(h, a_src, a_dst, hx_ref, as_ref, ad_ref, bbound_ref):
    hx_ref[...] = jnp.concatenate([h, jnp.ones((N, LANES), jnp.float32)], 1)
    as_ = jnp.dot(h, a_src, preferred_element_type=jnp.float32)
    ad_ = jnp.dot(h, a_dst, preferred_element_type=jnp.float32)
    pad = jnp.zeros((NPAD - N,), jnp.float32)
    as_ref[...] = jnp.broadcast_to(
        jnp.concatenate([as_, pad])[:, None], (NPAD, LANES))
    ad_ref[...] = jnp.broadcast_to(
        jnp.concatenate([ad_, pad])[:, None], (NPAD, LANES))
    m = jnp.max(as_) + jnp.max(ad_)
    b = jnp.where(m >= 0.0, m, 0.2 * m)
    bbound_ref[...] = jnp.full((LANES,), b, jnp.float32)


def _tc_pre_body(x_ref, w_ref, asrc_ref, adst_ref, hx_ref, as_ref, ad_ref,
                 bb_ref):
    h = jnp.dot(x_ref[...], w_ref[...], preferred_element_type=jnp.float32)
    _dense_head(h, asrc_ref[...], adst_ref[...], hx_ref, as_ref, ad_ref,
                bb_ref)


def _combine_bn_relu(acc_ref, b, gamma, beta):
    nm = acc_ref[0, pl.ds(0, N), pl.ds(0, D)] + acc_ref[1, pl.ds(0, N), pl.ds(0, D)]
    dn = acc_ref[0, pl.ds(0, N), D] + acc_ref[1, pl.ds(0, N), D]
    o = nm / (dn + 1e-16)[:, None] + b[None, :]
    mu = jnp.mean(o, axis=0)
    var = jnp.mean((o - mu[None, :]) ** 2, axis=0)
    xn = (o - mu[None, :]) / jnp.sqrt(var + 1e-5) * gamma[None, :] + beta[None, :]
    return jnp.maximum(xn, 0.0)


def _tc_mid_body(acc_ref, b_ref, g_ref, be_ref, w_ref, asrc_ref,
                 adst_ref, hx_ref, as_ref, ad_ref, bb_ref):
    x2 = _combine_bn_relu(acc_ref, b_ref[...], g_ref[...], be_ref[...])
    h = jnp.dot(x2, w_ref[...], preferred_element_type=jnp.float32)
    _dense_head(h, asrc_ref[...], adst_ref[...], hx_ref, as_ref, ad_ref,
                bb_ref)


def _tc_final_body(acc_ref, b_ref, g_ref, be_ref, batch_ref, out_ref):
    hf = _combine_bn_relu(acc_ref, b_ref[...], g_ref[...], be_ref[...])
    bi = batch_ref[...]
    gids = lax.broadcasted_iota(jnp.int32, (N, G), 1)
    oh = (bi == gids).astype(jnp.float32)
    pooled = lax.dot_general(oh, hf, (((0,), (0,)), ((), ())),
                             preferred_element_type=jnp.float32)
    counts = jnp.sum(oh, axis=0)
    out_ref[...] = pooled / jnp.maximum(counts, 1.0)[:, None]


def _sc_edge_body(nb, hx_hbm, as_hbm, ad_hbm, bb_hbm, src_hbm, dst_hbm,
                  acc_out,
                  acc_sh, srcb_v, dstb_v, rows_v, asr_v, adr_v, b_v, sem0):
    cid = lax.axis_index("c")
    sid = lax.axis_index("s")
    wid = cid * NS + sid

    pltpu.sync_copy(bb_hbm, b_v)

    zero16 = jnp.zeros((LANES,), jnp.float32)

    def zero_body(i, _):
        for k in range(DX // LANES):
            rows_v[i, pl.ds(k * LANES, LANES)] = zero16
        return 0

    lax.fori_loop(0, C, zero_body, 0)
    for r in range(RPT // C):
        pltpu.sync_copy(rows_v, acc_sh.at[pl.ds(sid * RPT + r * C, C)])
    plsc.subcore_barrier()

    bshift = b_v[...]

    def block_body(b, _):
        pltpu.sync_copy(src_hbm.at[wid, b], srcb_v)
        pltpu.sync_copy(dst_hbm.at[wid, b], dstb_v)
        for jj in range(BLK):
            ca = pltpu.async_copy(as_hbm.at[srcb_v.at[jj]], asr_v, sem0)
            cb = pltpu.async_copy(ad_hbm.at[dstb_v.at[jj]], adr_v, sem0)
            cr = pltpu.async_copy(hx_hbm.at[srcb_v.at[jj]], rows_v, sem0)
            ca.wait()
            cb.wait()
            cr.wait()

            @plsc.parallel_loop(0, C, unroll=4)
            def scale_body(i):
                a = asr_v[i, :] + adr_v[i, :]
                a = jnp.where(a >= 0.0, a, 0.2 * a)
                ev = jnp.exp(a - bshift)
                for k in range(DX // LANES):
                    rows_v[i, pl.ds(k * LANES, LANES)] = (
                        rows_v[i, pl.ds(k * LANES, LANES)] * ev)

            pltpu.sync_copy(rows_v, acc_sh.at[dstb_v.at[jj]], add=True)
        return 0

    lax.fori_loop(0, nb, block_body, 0)
    plsc.subcore_barrier()

    for r in range(RPT // C):
        pltpu.sync_copy(acc_sh.at[pl.ds(sid * RPT + r * C, C)], rows_v)
        pltpu.sync_copy(rows_v, acc_out.at[cid, pl.ds(sid * RPT + r * C, C)])


def _sc_edge_pass(hx, as_, ad_, bb, src2d, dst2d, nb):
    mesh = plsc.VectorSubcoreMesh(core_axis_name="c", subcore_axis_name="s",
                                  num_cores=NC, num_subcores=NS)
    kern = pl.kernel(
        functools.partial(_sc_edge_body, nb),
        out_type=jax.ShapeDtypeStruct((NC, NPAD, DX), jnp.float32),
        mesh=mesh,
        compiler_params=pltpu.CompilerParams(needs_layout_passes=False,
                                             use_tc_tiling_on_sc=False),
        scratch_types=[
            pltpu.VMEM_SHARED((NPAD, DX), jnp.float32),
            pltpu.VMEM((BLK, C), jnp.int32),
            pltpu.VMEM((BLK, C), jnp.int32),
            pltpu.VMEM((C, DX), jnp.float32),
            pltpu.VMEM((C, LANES), jnp.float32),
            pltpu.VMEM((C, LANES), jnp.float32),
            pltpu.VMEM((LANES,), jnp.float32),
            pltpu.SemaphoreType.DMA,
        ],
    )
    return kern(hx, as_, ad_, bb, src2d, dst2d)


def kernel(x, params, edge_index, batch):
    e = edge_index.shape[1]
    etot = e + N
    nb = -(-etot // (NW * C * BLK))
    epad = NW * nb * BLK * C - etot

    loop = jnp.arange(N, dtype=jnp.int32)
    src2d = jnp.concatenate(
        [edge_index[0], loop, jnp.zeros((epad,), jnp.int32)]
    ).reshape(NW, nb, BLK, C)
    dst2d = jnp.concatenate(
        [edge_index[1], loop, jnp.full((epad,), N, jnp.int32)]
    ).reshape(NW, nb, BLK, C)
    batch2 = batch.reshape(N, 1)

    tc_pre = pl.pallas_call(
        _tc_pre_body,
        out_shape=[
            jax.ShapeDtypeStruct((N, DX), jnp.float32),
            jax.ShapeDtypeStruct((NPAD, LANES), jnp.float32),
            jax.ShapeDtypeStruct((NPAD, LANES), jnp.float32),
            jax.ShapeDtypeStruct((LANES,), jnp.float32),
        ],
    )
    tc_mid = pl.pallas_call(
        _tc_mid_body,
        out_shape=[
            jax.ShapeDtypeStruct((N, DX), jnp.float32),
            jax.ShapeDtypeStruct((NPAD, LANES), jnp.float32),
            jax.ShapeDtypeStruct((NPAD, LANES), jnp.float32),
            jax.ShapeDtypeStruct((LANES,), jnp.float32),
        ],
    )
    tc_final = pl.pallas_call(
        _tc_final_body,
        out_shape=jax.ShapeDtypeStruct((G, D), jnp.float32),
    )

    p = params
    hx1, as1, ad1, bb1 = tc_pre(x, p['W1'], p['a_src1'], p['a_dst1'])
    acc1 = _sc_edge_pass(hx1, as1, ad1, bb1, src2d, dst2d, nb)
    hx2, as2, ad2, bb2 = tc_mid(acc1, p['b1'], p['gamma1'], p['beta1'],
                                p['W2'], p['a_src2'], p['a_dst2'])
    acc2 = _sc_edge_pass(hx2, as2, ad2, bb2, src2d, dst2d, nb)
    return tc_final(acc2, p['b2'], p['gamma2'], p['beta2'], batch2)
